# speedup vs baseline: 10.7256x; 10.7256x over previous
import functools

import jax
import jax.numpy as jnp
from jax import lax
from jax.experimental import pallas as pl
from jax.experimental.pallas import tpu as pltpu
from jax.experimental.pallas import tpu_sc as plsc

_EPS = 1e-12


def _sc_gather(word_emb, ids_blocks):
    NW, NCHUNK, CL = ids_blocks.shape
    V, D = word_emb.shape
    N = NW * NCHUNK * CL

    info = plsc.get_sparse_core_info()
    NC = info.num_cores

    mesh = plsc.VectorSubcoreMesh(core_axis_name="c", subcore_axis_name="s")

    @functools.partial(
        pl.kernel,
        mesh=mesh,
        out_type=jax.ShapeDtypeStruct((N, D), jnp.float32),
        scratch_types=[
            pltpu.VMEM((NCHUNK, CL), jnp.int32),
            pltpu.VMEM((CL, D), jnp.float32),
            pltpu.SemaphoreType.DMA,
        ],
    )
    def k(table_hbm, idx_hbm, out_hbm, idx_v, rows, sem):
        wid = lax.axis_index("s") * NC + lax.axis_index("c")
        base = wid * (NCHUNK * CL)
        pltpu.sync_copy(idx_hbm.at[wid], idx_v)

        def body(j, carry):
            pltpu.async_copy(table_hbm.at[idx_v.at[j]], rows, sem).wait()
            pltpu.sync_copy(rows, out_hbm.at[pl.ds(base + j * CL, CL)])
            return carry

        lax.fori_loop(0, NCHUNK, body, 0)

    return k(word_emb, ids_blocks)


def _tc_layernorm(gathered, pos_emb, type_emb, ts_emb, gamma, beta, block_b):
    B, L, D = gathered.shape

    def body(g_ref, pos_ref, type_ref, ts_ref, gamma_ref, beta_ref, out_ref):
        bias = pos_ref[...] + type_ref[0:1, :] + ts_ref[1:2, :]
        x = g_ref[...] + bias[None, :, :]
        mean = jnp.mean(x, axis=-1, keepdims=True)
        xc = x - mean
        var = jnp.mean(xc * xc, axis=-1, keepdims=True)
        y = xc * lax.rsqrt(var + _EPS)
        out_ref[...] = y * gamma_ref[...] + beta_ref[...]

    return pl.pallas_call(
        body,
        grid=(B // block_b,),
        in_specs=[
            pl.BlockSpec((block_b, L, D), lambda i: (i, 0, 0)),
            pl.BlockSpec((L, D), lambda i: (0, 0)),
            pl.BlockSpec(type_emb.shape, lambda i: (0, 0)),
            pl.BlockSpec(ts_emb.shape, lambda i: (0, 0)),
            pl.BlockSpec((D,), lambda i: (0,)),
            pl.BlockSpec((D,), lambda i: (0,)),
        ],
        out_specs=pl.BlockSpec((block_b, L, D), lambda i: (i, 0, 0)),
        out_shape=jax.ShapeDtypeStruct((B, L, D), jnp.float32),
    )(gathered, pos_emb, type_emb, ts_emb, gamma, beta)


def kernel(input_ids, word_emb, pos_emb, type_emb, ts_emb, gamma, beta):
    B, L = input_ids.shape
    V, D = word_emb.shape
    n = B * L
    NW = 32
    CL = 128
    assert n % (NW * CL) == 0
    nchunk = n // (NW * CL)
    ids_blocks = input_ids.reshape(NW, nchunk, CL)
    gathered = _sc_gather(word_emb, ids_blocks).reshape(B, L, D)
    return _tc_layernorm(gathered, pos_emb, type_emb, ts_emb, gamma, beta, block_b=32)


# speedup vs baseline: 12.7381x; 1.1876x over previous
import functools

import jax
import jax.numpy as jnp
from jax import lax
from jax.experimental import pallas as pl
from jax.experimental.pallas import tpu as pltpu
from jax.experimental.pallas import tpu_sc as plsc

_EPS = 1e-12


def _sc_gather(word_emb, ids_blocks):
    NW, NCHUNK, CL = ids_blocks.shape
    V, D = word_emb.shape
    N = NW * NCHUNK * CL

    info = plsc.get_sparse_core_info()
    NC = info.num_cores

    mesh = plsc.VectorSubcoreMesh(core_axis_name="c", subcore_axis_name="s")

    @functools.partial(
        pl.kernel,
        mesh=mesh,
        out_type=jax.ShapeDtypeStruct((N, D), jnp.float32),
        scratch_types=[
            pltpu.VMEM((NCHUNK, CL), jnp.int32),
            pltpu.VMEM((CL, D), jnp.float32),
            pltpu.VMEM((CL, D), jnp.float32),
            pltpu.SemaphoreType.DMA,
            pltpu.SemaphoreType.DMA,
            pltpu.SemaphoreType.DMA,
            pltpu.SemaphoreType.DMA,
        ],
    )
    def k(table_hbm, idx_hbm, out_hbm, idx_v, rows0, rows1, g0, g1, s0, s1):
        wid = lax.axis_index("s") * NC + lax.axis_index("c")
        base = wid * (NCHUNK * CL)
        pltpu.sync_copy(idx_hbm.at[wid], idx_v)

        def gather(j, buf, sem):
            pltpu.async_copy(table_hbm.at[idx_v.at[j]], buf, sem)

        def scatter(j, buf, sem):
            pltpu.async_copy(buf, out_hbm.at[pl.ds(base + j * CL, CL)], sem)

        def wait_s(buf, sem):
            pltpu.make_async_copy(buf, out_hbm.at[pl.ds(base, CL)], sem).wait()

        def wait_g(buf, sem):
            pltpu.make_async_copy(table_hbm.at[idx_v.at[0]], buf, sem).wait()

        gather(0, rows0, g0)
        n2 = NCHUNK // 2

        def body(i, carry):
            j0 = 2 * i

            @pl.when(i > 0)
            def _():
                wait_s(rows1, s1)

            gather(j0 + 1, rows1, g1)
            wait_g(rows0, g0)
            scatter(j0, rows0, s0)

            @pl.when(i + 1 < n2)
            def _():
                wait_s(rows0, s0)
                gather(j0 + 2, rows0, g0)

            wait_g(rows1, g1)
            scatter(j0 + 1, rows1, s1)
            return carry

        lax.fori_loop(0, n2, body, 0)
        wait_s(rows0, s0)
        wait_s(rows1, s1)

    return k(word_emb, ids_blocks)


def _tc_layernorm(gathered, pos_emb, type_emb, ts_emb, gamma, beta, block_b):
    B, L, D = gathered.shape

    def body(g_ref, pos_ref, type_ref, ts_ref, gamma_ref, beta_ref, out_ref):
        bias = pos_ref[...] + type_ref[0:1, :] + ts_ref[1:2, :]
        x = g_ref[...] + bias[None, :, :]
        mean = jnp.mean(x, axis=-1, keepdims=True)
        xc = x - mean
        var = jnp.mean(xc * xc, axis=-1, keepdims=True)
        y = xc * lax.rsqrt(var + _EPS)
        out_ref[...] = y * gamma_ref[...] + beta_ref[...]

    return pl.pallas_call(
        body,
        grid=(B // block_b,),
        in_specs=[
            pl.BlockSpec((block_b, L, D), lambda i: (i, 0, 0)),
            pl.BlockSpec((L, D), lambda i: (0, 0)),
            pl.BlockSpec(type_emb.shape, lambda i: (0, 0)),
            pl.BlockSpec(ts_emb.shape, lambda i: (0, 0)),
            pl.BlockSpec((D,), lambda i: (0,)),
            pl.BlockSpec((D,), lambda i: (0,)),
        ],
        out_specs=pl.BlockSpec((block_b, L, D), lambda i: (i, 0, 0)),
        out_shape=jax.ShapeDtypeStruct((B, L, D), jnp.float32),
    )(gathered, pos_emb, type_emb, ts_emb, gamma, beta)


def kernel(input_ids, word_emb, pos_emb, type_emb, ts_emb, gamma, beta):
    B, L = input_ids.shape
    V, D = word_emb.shape
    n = B * L
    NW = 32
    CL = 128
    assert n % (NW * CL) == 0
    nchunk = n // (NW * CL)
    ids_blocks = input_ids.reshape(NW, nchunk, CL)
    gathered = _sc_gather(word_emb, ids_blocks).reshape(B, L, D)
    return _tc_layernorm(gathered, pos_emb, type_emb, ts_emb, gamma, beta, block_b=32)


# speedup vs baseline: 13.3976x; 1.0518x over previous
import functools

import jax
import jax.numpy as jnp
from jax import lax
from jax.experimental import pallas as pl
from jax.experimental.pallas import tpu as pltpu
from jax.experimental.pallas import tpu_sc as plsc

_EPS = 1e-12


def _sc_gather(word_emb, ids_blocks):
    NW, NCHUNK, CL = ids_blocks.shape
    V, D = word_emb.shape
    N = NW * NCHUNK * CL

    info = plsc.get_sparse_core_info()
    NC = info.num_cores

    mesh = plsc.VectorSubcoreMesh(core_axis_name="c", subcore_axis_name="s")

    @functools.partial(
        pl.kernel,
        mesh=mesh,
        out_type=jax.ShapeDtypeStruct((N, D), jnp.float32),
        scratch_types=[
            pltpu.VMEM((NCHUNK, CL), jnp.int32),
            pltpu.VMEM((CL, D), jnp.float32),
            pltpu.VMEM((CL, D), jnp.float32),
            pltpu.SemaphoreType.DMA,
            pltpu.SemaphoreType.DMA,
            pltpu.SemaphoreType.DMA,
            pltpu.SemaphoreType.DMA,
        ],
    )
    def k(table_hbm, idx_hbm, out_hbm, idx_v, rows0, rows1, g0, g1, s0, s1):
        wid = lax.axis_index("s") * NC + lax.axis_index("c")
        base = wid * (NCHUNK * CL)
        pltpu.sync_copy(idx_hbm.at[wid], idx_v)

        def gather(j, buf, sem):
            pltpu.async_copy(table_hbm.at[idx_v.at[j]], buf, sem)

        def scatter(j, buf, sem):
            pltpu.async_copy(buf, out_hbm.at[pl.ds(base + j * CL, CL)], sem)

        def wait_s(buf, sem):
            pltpu.make_async_copy(buf, out_hbm.at[pl.ds(base, CL)], sem).wait()

        def wait_g(buf, sem):
            pltpu.make_async_copy(table_hbm.at[idx_v.at[0]], buf, sem).wait()

        gather(0, rows0, g0)
        n2 = NCHUNK // 2

        def body(i, carry):
            j0 = 2 * i

            @pl.when(i > 0)
            def _():
                wait_s(rows1, s1)

            gather(j0 + 1, rows1, g1)
            wait_g(rows0, g0)
            scatter(j0, rows0, s0)

            @pl.when(i + 1 < n2)
            def _():
                wait_s(rows0, s0)
                gather(j0 + 2, rows0, g0)

            wait_g(rows1, g1)
            scatter(j0 + 1, rows1, s1)
            return carry

        lax.fori_loop(0, n2, body, 0)
        wait_s(rows0, s0)
        wait_s(rows1, s1)

    return k(word_emb, ids_blocks)


def _tc_layernorm(gathered, pos_emb, type_emb, ts_emb, gamma, beta, block_b):
    B, L, D = gathered.shape

    def body(g_ref, pos_ref, type_ref, ts_ref, gamma_ref, beta_ref, out_ref):
        bias = pos_ref[...] + type_ref[0:1, :] + ts_ref[1:2, :]
        x = g_ref[...] + bias[None, :, :]
        mean = jnp.mean(x, axis=-1, keepdims=True)
        xc = x - mean
        var = jnp.mean(xc * xc, axis=-1, keepdims=True)
        y = xc * lax.rsqrt(var + _EPS)
        out_ref[...] = y * gamma_ref[...] + beta_ref[...]

    return pl.pallas_call(
        body,
        grid=(B // block_b,),
        in_specs=[
            pl.BlockSpec((block_b, L, D), lambda i: (i, 0, 0)),
            pl.BlockSpec((L, D), lambda i: (0, 0)),
            pl.BlockSpec(type_emb.shape, lambda i: (0, 0)),
            pl.BlockSpec(ts_emb.shape, lambda i: (0, 0)),
            pl.BlockSpec((D,), lambda i: (0,)),
            pl.BlockSpec((D,), lambda i: (0,)),
        ],
        out_specs=pl.BlockSpec((block_b, L, D), lambda i: (i, 0, 0)),
        out_shape=jax.ShapeDtypeStruct((B, L, D), jnp.float32),
    )(gathered, pos_emb, type_emb, ts_emb, gamma, beta)


def kernel(input_ids, word_emb, pos_emb, type_emb, ts_emb, gamma, beta):
    B, L = input_ids.shape
    V, D = word_emb.shape
    n = B * L
    NW = 32
    CL = 128
    assert n % (NW * CL) == 0
    nchunk = n // (NW * CL)
    ids_blocks = input_ids.reshape(NW, nchunk, CL)
    gathered = _sc_gather(word_emb, ids_blocks).reshape(B, L, D)
    return _tc_layernorm(gathered, pos_emb, type_emb, ts_emb, gamma, beta, block_b=64)
